# baseline (speedup 1.0000x reference)
import jax
import jax.numpy as jnp
from jax import lax
from jax.experimental import pallas as pl
from jax.experimental.pallas import tpu as pltpu
from jax.experimental.pallas import tpu_sc as plsc

_NW = 32
_W = 128
_PER = 2
_CH = _PER * _W


def _gather_body(table_hbm, ids_hbm, out_hbm, idx_v, buf0, buf1, sg, so0, so1):
    w = lax.axis_index("c") * 16 + lax.axis_index("s")
    rows_w = out_hbm.shape[0] // _NW
    n_chunks = rows_w // _CH
    base = w * rows_w

    pltpu.sync_copy(ids_hbm.at[w], idx_v)

    bufs = (buf0, buf1)
    sems = (so0, so1)

    @pl.loop(0, n_chunks, step=2)
    def _(g):
        for b in range(2):
            buf = bufs[b]
            so = sems[b]
            c = g + b

            @pl.when(c >= 2)
            def _():
                pltpu.make_async_copy(
                    buf, out_hbm.at[pl.ds(base, _CH)], so
                ).wait()

            for j in range(_PER):
                pltpu.async_copy(
                    table_hbm.at[idx_v.at[c * _PER + j]],
                    buf.at[pl.ds(j * _W, _W), :],
                    sg,
                ).wait()

            pltpu.async_copy(buf, out_hbm.at[pl.ds(base + c * _CH, _CH)], so)

    pltpu.make_async_copy(buf0, out_hbm.at[pl.ds(base, _CH)], so0).wait()
    pltpu.make_async_copy(buf1, out_hbm.at[pl.ds(base, _CH)], so1).wait()


def kernel(input_ids, r_table, g_table, b_table):
    b, t, c = input_ids.shape
    v, d = r_table.shape
    n = b * t * c
    rows_w = n // _NW

    table = jnp.concatenate([r_table, g_table, b_table], axis=0)
    offsets = jnp.arange(c, dtype=input_ids.dtype) * v
    flat_ids = (input_ids + offsets).reshape(_NW, rows_w // _W, _W)

    mesh = plsc.VectorSubcoreMesh(core_axis_name="c", subcore_axis_name="s")
    gather = pl.kernel(
        _gather_body,
        out_type=jax.ShapeDtypeStruct((n, d), jnp.float32),
        mesh=mesh,
        scratch_types=[
            pltpu.VMEM((rows_w // _W, _W), jnp.int32),
            pltpu.VMEM((_CH, d), jnp.float32),
            pltpu.VMEM((_CH, d), jnp.float32),
            pltpu.SemaphoreType.DMA,
            pltpu.SemaphoreType.DMA,
            pltpu.SemaphoreType.DMA,
        ],
    )
    out = gather(table, flat_ids)
    return out.reshape(b, t, c * d)


# speedup vs baseline: 1.6910x; 1.6910x over previous
import jax
import jax.numpy as jnp
from jax import lax
from jax.experimental import pallas as pl
from jax.experimental.pallas import tpu as pltpu
from jax.experimental.pallas import tpu_sc as plsc

_NW = 32
_W = 128
_TCH = 128


def _gather_body(table_hbm, ids_hbm, out_hbm, idx_v, buf0, buf1, sg, so0, so1):
    w = lax.axis_index("c") * 16 + lax.axis_index("s")
    tokens = out_hbm.shape[0]
    tok_w = tokens // _NW
    n_chunks = tok_w // _TCH
    tok_base = w * tok_w
    d_out = out_hbm.shape[1]

    pltpu.sync_copy(ids_hbm.at[w], idx_v)

    bufs = (buf0, buf1)
    sems = (so0, so1)

    @pl.loop(0, n_chunks, step=2)
    def _(g):
        for b in range(2):
            buf = bufs[b]
            so = sems[b]
            c = g + b

            @pl.when(c >= 2)
            def _():
                pltpu.make_async_copy(
                    buf.reshape(_TCH, d_out),
                    out_hbm.at[pl.ds(tok_base, _TCH), :],
                    so,
                ).wait()

            for j in range(3):
                pltpu.async_copy(
                    table_hbm.at[idx_v.at[c * 3 + j]],
                    buf.at[pl.ds(j * _W, _W), :],
                    sg,
                ).wait()

            pltpu.async_copy(
                buf.reshape(_TCH, d_out),
                out_hbm.at[pl.ds(tok_base + c * _TCH, _TCH), :],
                so,
            )

    for buf, so in ((buf0, so0), (buf1, so1)):
        pltpu.make_async_copy(
            buf.reshape(_TCH, d_out),
            out_hbm.at[pl.ds(tok_base, _TCH), :],
            so,
        ).wait()


def kernel(input_ids, r_table, g_table, b_table):
    b, t, c = input_ids.shape
    v, d = r_table.shape
    tokens = b * t
    n = tokens * c
    rows_w = n // _NW

    table = jnp.concatenate([r_table, g_table, b_table], axis=0)
    offsets = jnp.arange(c, dtype=input_ids.dtype) * v
    flat_ids = (input_ids + offsets).reshape(_NW, rows_w // _W, _W)

    mesh = plsc.VectorSubcoreMesh(core_axis_name="c", subcore_axis_name="s")
    gather = pl.kernel(
        _gather_body,
        out_type=jax.ShapeDtypeStruct((tokens, c * d), jnp.float32),
        mesh=mesh,
        scratch_types=[
            pltpu.VMEM((rows_w // _W, _W), jnp.int32),
            pltpu.VMEM((3 * _W, d), jnp.float32),
            pltpu.VMEM((3 * _W, d), jnp.float32),
            pltpu.SemaphoreType.DMA,
            pltpu.SemaphoreType.DMA,
            pltpu.SemaphoreType.DMA,
        ],
    )
    out = gather(table, flat_ids)
    return out.reshape(b, t, c * d)
